# baseline (speedup 1.0000x reference)
import functools

import jax
import jax.numpy as jnp
from jax.experimental import pallas as pl

K_CHUNK = 128
ROW_CHUNK = 256


def _vq_kernel(x_ref, w_ref, q_ref, idx_ref):
    n_rows = x_ref.shape[1]
    d = x_ref.shape[2]
    k_total = w_ref.shape[1]

    best_i_parts = []
    for r0 in range(0, n_rows, ROW_CHUNK):
        xt = x_ref[0, r0:r0 + ROW_CHUNK, :]
        best_v = jnp.full((ROW_CHUNK,), jnp.inf, dtype=jnp.float32)
        best_i = jnp.zeros((ROW_CHUNK,), dtype=jnp.int32)
        for kc in range(0, k_total, K_CHUNK):
            wchunk = w_ref[:, kc:kc + K_CHUNK]
            acc = jnp.zeros((ROW_CHUNK, K_CHUNK), dtype=jnp.float32)
            for j in range(d):
                t = xt[:, j:j + 1] - wchunk[j:j + 1, :]
                acc = acc + t * t
            cmin = jnp.min(acc, axis=1)
            lane = jax.lax.broadcasted_iota(jnp.int32, (ROW_CHUNK, K_CHUNK), 1)
            cidx = jnp.min(
                jnp.where(acc == cmin[:, None], lane, K_CHUNK), axis=1)
            better = cmin < best_v
            best_v = jnp.where(better, cmin, best_v)
            best_i = jnp.where(better, kc + cidx, best_i)
        best_i_parts.append(best_i)

    idx_all = jnp.concatenate(best_i_parts)
    idx_ref[0, 0, :] = idx_all

    onehot = (jax.lax.broadcasted_iota(jnp.int32, (n_rows, k_total), 1)
              == idx_all[:, None]).astype(jnp.bfloat16)
    w_f32 = w_ref[...]
    w_hi = w_f32.astype(jnp.bfloat16)
    rem = w_f32 - w_hi.astype(jnp.float32)
    w_mid = rem.astype(jnp.bfloat16)
    w_lo = (rem - w_mid.astype(jnp.float32)).astype(jnp.bfloat16)
    parts = []
    for wp in (w_hi, w_mid, w_lo):
        parts.append(jax.lax.dot_general(
            wp, onehot,
            dimension_numbers=(((1,), (1,)), ((), ())),
            preferred_element_type=jnp.float32))
    q_ref[0] = (parts[0] + parts[1]) + parts[2]


@jax.jit
def kernel(x, weight):
    b, d, h, w = x.shape
    k = weight.shape[1]
    rows = h * w
    xt = jnp.transpose(x, (0, 2, 3, 1)).reshape(b, rows, d)

    q, idx = pl.pallas_call(
        _vq_kernel,
        grid=(b,),
        in_specs=[
            pl.BlockSpec((1, rows, d), lambda i: (i, 0, 0)),
            pl.BlockSpec((d, k), lambda i: (0, 0)),
        ],
        out_specs=[
            pl.BlockSpec((1, d, rows), lambda i: (i, 0, 0)),
            pl.BlockSpec((1, 1, rows), lambda i: (i, 0, 0)),
        ],
        out_shape=[
            jax.ShapeDtypeStruct((b, d, rows), jnp.float32),
            jax.ShapeDtypeStruct((b, 1, rows), jnp.int32),
        ],
    )(xt, weight)

    return q.reshape(b, d, h, w), idx.reshape(b, h, w)


# speedup vs baseline: 1.2644x; 1.2644x over previous
import jax
import jax.numpy as jnp
from jax import lax
from jax.experimental import pallas as pl

K_CHUNK = 32


def _vq_kernel(x_ref, wt_ref, q_ref, idx_ref):
    d = x_ref.shape[1]
    n_rows = x_ref.shape[2]
    k_total = wt_ref.shape[0]
    n_chunks = k_total // K_CHUNK

    def chunk_body(c, carry):
        best_v, best_i = carry
        kc = c * K_CHUNK
        wc = wt_ref[pl.ds(kc, K_CHUNK), :]
        acc = jnp.zeros((K_CHUNK, n_rows), dtype=jnp.float32)
        for j in range(d):
            xrow = x_ref[0, j:j + 1, :]
            wcol = wc[:, j:j + 1]
            t = xrow - wcol
            acc = acc + t * t
        cmin = jnp.min(acc, axis=0, keepdims=True)
        kiota = jax.lax.broadcasted_iota(
            jnp.int32, (K_CHUNK, n_rows), 0)
        cidx = jnp.min(
            jnp.where(acc == cmin, kiota, K_CHUNK), axis=0, keepdims=True)
        better = cmin < best_v
        best_v = jnp.where(better, cmin, best_v)
        best_i = jnp.where(better, kc + cidx, best_i)
        return best_v, best_i

    best_v = jnp.full((1, n_rows), jnp.inf, dtype=jnp.float32)
    best_i = jnp.zeros((1, n_rows), dtype=jnp.int32)
    best_v, best_i = lax.fori_loop(
        0, n_chunks, chunk_body, (best_v, best_i), unroll=False)

    idx_ref[0] = best_i

    onehot = (jax.lax.broadcasted_iota(jnp.int32, (k_total, n_rows), 0)
              == best_i).astype(jnp.bfloat16)
    wt_f32 = wt_ref[...]
    wt_hi = wt_f32.astype(jnp.bfloat16)
    rem = wt_f32 - wt_hi.astype(jnp.float32)
    wt_mid = rem.astype(jnp.bfloat16)
    wt_lo = (rem - wt_mid.astype(jnp.float32)).astype(jnp.bfloat16)
    parts = []
    for wp in (wt_hi, wt_mid, wt_lo):
        parts.append(jax.lax.dot_general(
            wp, onehot,
            dimension_numbers=(((0,), (0,)), ((), ())),
            preferred_element_type=jnp.float32))
    q_ref[0] = (parts[0] + parts[1]) + parts[2]


@jax.jit
def kernel(x, weight):
    b, d, h, w = x.shape
    k = weight.shape[1]
    rows = h * w
    xr = x.reshape(b, d, rows)
    wt = jnp.transpose(weight, (1, 0))

    q, idx = pl.pallas_call(
        _vq_kernel,
        grid=(b,),
        in_specs=[
            pl.BlockSpec((1, d, rows), lambda i: (i, 0, 0)),
            pl.BlockSpec((k, d), lambda i: (0, 0)),
        ],
        out_specs=[
            pl.BlockSpec((1, d, rows), lambda i: (i, 0, 0)),
            pl.BlockSpec((1, 1, rows), lambda i: (i, 0, 0)),
        ],
        out_shape=[
            jax.ShapeDtypeStruct((b, d, rows), jnp.float32),
            jax.ShapeDtypeStruct((b, 1, rows), jnp.int32),
        ],
    )(xr, wt)

    return q.reshape(b, d, h, w), idx.reshape(b, h, w)
